# Initial kernel scaffold; baseline (speedup 1.0000x reference)
import jax
import jax.numpy as jnp
from jax.experimental import pallas as pl


def kernel(features, indices, W):
    raise NotImplementedError("write your pallas kernel here")


# speedup vs baseline: 5.6569x; 5.6569x over previous
import functools

import jax
import jax.numpy as jnp
from jax import lax
from jax.experimental import pallas as pl
from jax.experimental.pallas import tpu as pltpu
from jax.experimental.pallas import tpu_sc as plsc

_N = 100000
_C = 64
_B = 2
_G = 128
_GP = _G + 4
_NPAD = 102400
_ROWS = _NPAD // 128
_TSIZE = 4_608_000
_PSAFE = ((0 * _GP + 2) * _GP + 2) * _GP + 2
_DMAX = _PSAFE

_NC = 2
_NS = 16


def _prep_body(f_ref, b_ref, z_ref, y_ref, x_ref,
               fa_ref, fm_ref, ps_ref, pg_ref):
  i = pl.program_id(0)
  f = f_ref[...]
  fa_ref[...] = jnp.mean(f, axis=1).reshape(8, 128)
  fm_ref[...] = jnp.max(f, axis=1).reshape(8, 128)
  r = lax.broadcasted_iota(jnp.int32, (8, 128), 0)
  c = lax.broadcasted_iota(jnp.int32, (8, 128), 1)
  vid = (i * 8 + r) * 128 + c
  p = ((b_ref[...] * _GP + z_ref[...] + 2) * _GP
       + y_ref[...] + 2) * _GP + x_ref[...] + 2
  valid = vid < _N
  ps_ref[...] = jnp.where(valid, p, 0)
  pg_ref[...] = jnp.where(valid, p, _PSAFE)


def _prep(feats_pad, b2, z2, y2, x2):
  coord_spec = pl.BlockSpec((8, 128), lambda i: (i, 0))
  return pl.pallas_call(
      _prep_body,
      grid=(100,),
      in_specs=[pl.BlockSpec((1024, 64), lambda i: (i, 0)),
                coord_spec, coord_spec, coord_spec, coord_spec],
      out_specs=[coord_spec, coord_spec, coord_spec, coord_spec],
      out_shape=[
          jax.ShapeDtypeStruct((_ROWS, 128), jnp.float32),
          jax.ShapeDtypeStruct((_ROWS, 128), jnp.float32),
          jax.ShapeDtypeStruct((_ROWS, 128), jnp.int32),
          jax.ShapeDtypeStruct((_ROWS, 128), jnp.int32),
      ],
  )(feats_pad, b2, z2, y2, x2)


_ZCHUNK = 12000


def _scatter_body(ps_hbm, fa_hbm, fm_hbm, ga_hbm, gm_hbm,
                  zbuf, idx_v, fa_v, fm_v, sem):
  tid = lax.axis_index("s")
  nv = _NPAD // _NS

  def zfill(t, carry):
    zbuf[pl.ds(16 * t, 16)] = jnp.zeros((16,), jnp.float32)
    return carry
  lax.fori_loop(0, _ZCHUNK // 16, zfill, 0)

  base = tid * (_TSIZE // _NS)

  def zdma(t, carry):
    pltpu.sync_copy(zbuf, ga_hbm.at[pl.ds(base + t * _ZCHUNK, _ZCHUNK)])
    pltpu.sync_copy(zbuf, gm_hbm.at[pl.ds(base + t * _ZCHUNK, _ZCHUNK)])
    return carry
  lax.fori_loop(0, (_TSIZE // _NS) // _ZCHUNK, zdma, 0)

  plsc.subcore_barrier()

  v0 = tid * nv
  pltpu.sync_copy(ps_hbm.at[pl.ds(v0, nv)], idx_v)
  pltpu.sync_copy(fa_hbm.at[pl.ds(v0, nv)], fa_v)
  pltpu.sync_copy(fm_hbm.at[pl.ds(v0, nv)], fm_v)
  ca = pltpu.async_copy(fa_v, ga_hbm.at[idx_v], sem)
  cm = pltpu.async_copy(fm_v, gm_hbm.at[idx_v], sem)
  ca.wait()
  cm.wait()


def _scatter(ps1, fa1, fm1):
  mesh = plsc.VectorSubcoreMesh(
      core_axis_name="c", subcore_axis_name="s", num_cores=1)
  nv = _NPAD // _NS
  return pl.kernel(
      _scatter_body,
      out_type=[jax.ShapeDtypeStruct((_TSIZE,), jnp.float32),
                jax.ShapeDtypeStruct((_TSIZE,), jnp.float32)],
      mesh=mesh,
      scratch_types=[
          pltpu.VMEM((_ZCHUNK,), jnp.float32),
          pltpu.VMEM((nv,), jnp.int32),
          pltpu.VMEM((nv,), jnp.float32),
          pltpu.VMEM((nv,), jnp.float32),
          pltpu.SemaphoreType.DMA,
      ],
  )(ps1, fa1, fm1)


def _gather_body(ga_hbm, gm_hbm, pg_hbm, w0_hbm, w1_hbm, acc_hbm,
                 pb_v, idx_v, ga_v, gm_v, acc_v, w0_v, w1_v, sem):
  wid = lax.axis_index("s") * _NC + lax.axis_index("c")
  nv = _NPAD // (_NC * _NS)
  ng = nv // 16
  v0 = wid * nv
  pltpu.sync_copy(pg_hbm.at[pl.ds(v0, nv)], pb_v)
  pltpu.sync_copy(w0_hbm, w0_v)
  pltpu.sync_copy(w1_hbm, w1_v)

  def azero(t, carry):
    acc_v[pl.ds(16 * t, 16)] = jnp.zeros((16,), jnp.float32)
    return carry
  lax.fori_loop(0, ng, azero, 0)

  def kbody(k, carry):
    dz = k // 25 - 2
    dy = (k // 5) % 5 - 2
    dx = k % 5 - 2
    d = (dz * _GP + dy) * _GP + dx

    def tbody(t, c2):
      s = pl.ds(16 * t, 16)
      idx_v[s] = pb_v[s] + d
      return c2
    lax.fori_loop(0, ng, tbody, 0)

    ca = pltpu.async_copy(ga_hbm.at[idx_v], ga_v, sem)
    cm = pltpu.async_copy(gm_hbm.at[idx_v], gm_v, sem)
    w0 = w0_v[k]
    w1 = w1_v[k]
    ca.wait()
    cm.wait()

    def tb(t, c2):
      s = pl.ds(16 * t, 16)
      acc_v[s] = acc_v[s] + w0 * ga_v[s] + w1 * gm_v[s]
      return c2
    lax.fori_loop(0, ng, tb, 0)
    return carry
  lax.fori_loop(0, 125, kbody, 0)

  pltpu.sync_copy(acc_v, acc_hbm.at[pl.ds(v0, nv)])


def _gather(ga, gm, pg1, w0t, w1t):
  mesh = plsc.VectorSubcoreMesh(core_axis_name="c", subcore_axis_name="s")
  nv = _NPAD // (_NC * _NS)
  return pl.kernel(
      _gather_body,
      out_type=jax.ShapeDtypeStruct((_NPAD,), jnp.float32),
      mesh=mesh,
      scratch_types=[
          pltpu.VMEM((nv,), jnp.int32),
          pltpu.VMEM((nv,), jnp.int32),
          pltpu.VMEM((nv,), jnp.float32),
          pltpu.VMEM((nv,), jnp.float32),
          pltpu.VMEM((nv,), jnp.float32),
          pltpu.VMEM((128, 16), jnp.float32),
          pltpu.VMEM((128, 16), jnp.float32),
          pltpu.SemaphoreType.DMA,
      ],
  )(ga, gm, pg1, w0t, w1t)


def _gate_body(f_ref, a_ref, o_ref):
  g8 = 1.0 / (1.0 + jnp.exp(-a_ref[...]))
  ones = jnp.ones((1, _C), jnp.float32)
  for s in range(8):
    gcol = lax.dot_general(g8[s:s + 1, :], ones, (((0,), (0,)), ((), ())),
                           preferred_element_type=jnp.float32)
    rs = pl.ds(s * 128, 128)
    o_ref[rs, :] = f_ref[rs, :] * gcol


def _gate(feats_pad, acc2):
  return pl.pallas_call(
      _gate_body,
      grid=(100,),
      in_specs=[pl.BlockSpec((1024, 64), lambda i: (i, 0)),
                pl.BlockSpec((8, 128), lambda i: (i, 0))],
      out_specs=pl.BlockSpec((1024, 64), lambda i: (i, 0)),
      out_shape=jax.ShapeDtypeStruct((_NPAD, _C), jnp.float32),
  )(feats_pad, acc2)


def kernel(features, indices, W):
  n = features.shape[0]
  pad = _NPAD - n
  feats_pad = jnp.pad(features, ((0, pad), (0, 0)))
  b2 = jnp.pad(indices[:, 0], (0, pad)).reshape(_ROWS, 128)
  z2 = jnp.pad(indices[:, 1], (0, pad)).reshape(_ROWS, 128)
  y2 = jnp.pad(indices[:, 2], (0, pad)).reshape(_ROWS, 128)
  x2 = jnp.pad(indices[:, 3], (0, pad)).reshape(_ROWS, 128)

  w0t = jnp.pad(jnp.broadcast_to(W[:, 0, :], (125, 16)), ((0, 3), (0, 0)))
  w1t = jnp.pad(jnp.broadcast_to(W[:, 1, :], (125, 16)), ((0, 3), (0, 0)))

  fa2, fm2, ps2, pg2 = _prep(feats_pad, b2, z2, y2, x2)
  ga, gm = _scatter(ps2.reshape(-1), fa2.reshape(-1), fm2.reshape(-1))
  acc1 = _gather(ga, gm, pg2.reshape(-1), w0t, w1t)
  out = _gate(feats_pad, acc1.reshape(_ROWS, 128))
  return out[:n]
